# baseline (speedup 1.0000x reference)
import functools

import jax
import jax.numpy as jnp
from jax import lax
from jax.experimental import pallas as pl
from jax.experimental.pallas import tpu as pltpu
from jax.experimental.pallas import tpu_sc as plsc

NC = 2
NS = 16
NW = NC * NS
L = 16

BATCH = 16384
K = 32
BPW = BATCH // NW
CHUNK = 128
NCHUNK = BPW // CHUNK


def _gmf_body(uid_hbm, iid_hbm, pt_hbm, qt_hbm, out_hbm,
              uid_v, iid_v, pu_v, qi_v, o_v, sem):
    wid = lax.axis_index("s") * NC + lax.axis_index("c")
    base = wid * BPW

    pltpu.sync_copy(uid_hbm.at[pl.ds(base, BPW)], uid_v)
    pltpu.sync_copy(iid_hbm.at[pl.ds(base, BPW)], iid_v)

    def fire(k, carry):
        for c in range(NCHUNK):
            sl = pl.ds(c * CHUNK, CHUNK)
            pltpu.async_copy(pt_hbm.at[k].at[uid_v.at[sl]],
                             pu_v.at[k, sl], sem)
            pltpu.async_copy(qt_hbm.at[k].at[iid_v.at[sl]],
                             qi_v.at[k, sl], sem)
        return carry

    lax.fori_loop(0, K, fire, 0)

    pltpu.make_async_copy(pt_hbm.at[:, pl.ds(0, BPW)], pu_v, sem).wait()
    pltpu.make_async_copy(qt_hbm.at[:, pl.ds(0, BPW)], qi_v, sem).wait()

    def group(g, carry):
        sl = pl.ds(g * L, L)

        def kstep(k, acc):
            return acc + pu_v[k, sl] * qi_v[k, sl]

        acc = lax.fori_loop(0, K, kstep, jnp.zeros((L,), jnp.float32))
        o_v[sl] = acc
        return carry

    def group_outer(g, carry):
        return group(g, carry)

    lax.fori_loop(0, BPW // L, group_outer, 0)

    pltpu.sync_copy(o_v, out_hbm.at[pl.ds(base, BPW)])


def _gmf(user_ids, item_ids, P, Q):
    mesh = plsc.VectorSubcoreMesh(
        core_axis_name="c", subcore_axis_name="s",
        num_cores=NC, num_subcores=NS)
    run = pl.kernel(
        _gmf_body,
        out_type=jax.ShapeDtypeStruct((BATCH,), jnp.float32),
        mesh=mesh,
        compiler_params=pltpu.CompilerParams(
            needs_layout_passes=False, use_tc_tiling_on_sc=False),
        scratch_types=[
            pltpu.VMEM((BPW,), jnp.int32),
            pltpu.VMEM((BPW,), jnp.int32),
            pltpu.VMEM((K, BPW), jnp.float32),
            pltpu.VMEM((K, BPW), jnp.float32),
            pltpu.VMEM((BPW,), jnp.float32),
            pltpu.SemaphoreType.DMA,
        ],
    )
    return run(user_ids, item_ids,
               jnp.swapaxes(P, 0, 1), jnp.swapaxes(Q, 0, 1))


def kernel(user_ids, item_ids, P, Q):
    out = _gmf(user_ids, item_ids, P, Q)
    return out.reshape(BATCH, 1)


# speedup vs baseline: 8.5752x; 8.5752x over previous
import functools

import jax
import jax.numpy as jnp
from jax import lax
from jax.experimental import pallas as pl
from jax.experimental.pallas import tpu as pltpu
from jax.experimental.pallas import tpu_sc as plsc

NC = 2
NS = 16
NW = NC * NS
L = 16

BATCH = 16384
K = 32
BPW = BATCH // NW
HALF = BPW // 2


def _gmf_body(uid_hbm, iid_hbm, p_hbm, q_hbm, out_hbm,
              uid_v, iid_v, pu_v, qi_v, o_v, sem):
    wid = lax.axis_index("s") * NC + lax.axis_index("c")
    base = wid * BPW

    pltpu.sync_copy(uid_hbm.at[pl.ds(base, BPW)], uid_v)
    pltpu.sync_copy(iid_hbm.at[pl.ds(base, BPW)], iid_v)

    lane = lax.iota(jnp.int32, L)

    def half(h, carry):
        hbase = h * HALF

        def fire(g, c):
            uvec = uid_v[pl.ds(hbase + g * L, L)]
            ivec = iid_v[pl.ds(hbase + g * L, L)]
            for j in range(L):
                b = g * L + j
                pltpu.async_copy(p_hbm.at[uvec[j]], pu_v.at[b], sem)
                pltpu.async_copy(q_hbm.at[ivec[j]], qi_v.at[b], sem)
            return c

        lax.fori_loop(0, HALF // L, fire, 0)

        def drain(b, c):
            pltpu.make_async_copy(p_hbm.at[0], pu_v.at[b], sem).wait()
            pltpu.make_async_copy(q_hbm.at[0], qi_v.at[b], sem).wait()
            return c

        lax.fori_loop(0, HALF, drain, 0)

        def group(g, c):
            row = g * L + lane

            def col_step(t, acc):
                col = (lane + t) & (K - 1)
                pv = plsc.load_gather(pu_v, [row, col])
                qv = plsc.load_gather(qi_v, [row, col])
                return acc + pv * qv

            acc = lax.fori_loop(0, K, col_step, jnp.zeros((L,), jnp.float32))
            o_v[pl.ds(hbase + g * L, L)] = acc
            return c

        lax.fori_loop(0, HALF // L, group, 0)
        return carry

    lax.fori_loop(0, BPW // HALF, half, 0)

    pltpu.sync_copy(o_v, out_hbm.at[pl.ds(base, BPW)])


@jax.jit
def _gmf(user_ids, item_ids, P, Q):
    mesh = plsc.VectorSubcoreMesh(
        core_axis_name="c", subcore_axis_name="s",
        num_cores=NC, num_subcores=NS)
    run = pl.kernel(
        _gmf_body,
        out_type=jax.ShapeDtypeStruct((BATCH,), jnp.float32),
        mesh=mesh,
        compiler_params=pltpu.CompilerParams(needs_layout_passes=False),
        scratch_types=[
            pltpu.VMEM((BPW,), jnp.int32),
            pltpu.VMEM((BPW,), jnp.int32),
            pltpu.VMEM((HALF, K), jnp.float32),
            pltpu.VMEM((HALF, K), jnp.float32),
            pltpu.VMEM((BPW,), jnp.float32),
            pltpu.SemaphoreType.DMA,
        ],
    )
    return run(user_ids, item_ids, P, Q)


def kernel(user_ids, item_ids, P, Q):
    out = _gmf(user_ids, item_ids, P, Q)
    return out.reshape(BATCH, 1)


# speedup vs baseline: 20.9921x; 2.4480x over previous
import functools

import jax
import jax.numpy as jnp
from jax import lax
from jax.experimental import pallas as pl
from jax.experimental.pallas import tpu as pltpu
from jax.experimental.pallas import tpu_sc as plsc

NC = 2
NS = 16
NW = NC * NS
L = 16

BATCH = 16384
K = 32
TB = 128
BPW = BATCH // NW
SUB = 8


def _gmf_body(uid_hbm, iid_hbm, pt_hbm, qt_hbm, out_hbm,
              uid_v, iid_v, pu_v, qi_v, o_v, sem):
    wid = lax.axis_index("s") * NC + lax.axis_index("c")
    base = wid * BPW

    pltpu.sync_copy(uid_hbm.at[pl.ds(base, BPW)], uid_v.at[pl.ds(0, BPW)])
    pltpu.sync_copy(iid_hbm.at[pl.ds(base, BPW)], iid_v.at[pl.ds(0, BPW)])

    lane = lax.iota(jnp.int32, L)
    blk = jnp.bitwise_and(lane, SUB - 1)

    def pair16(g, carry):
        uvec = uid_v[pl.ds(g * L, L)]
        ivec = iid_v[pl.ds(g * L, L)]
        ucol = jnp.bitwise_and(uvec, TB - 1)
        icol = jnp.bitwise_and(ivec, TB - 1)
        acc = jnp.zeros((L,), jnp.float32)

        for half in range(2):
            for j in range(SUB):
                u = pl.multiple_of((uvec[half * SUB + j] >> 7) << 7, TB)
                i = pl.multiple_of((ivec[half * SUB + j] >> 7) << 7, TB)
                pltpu.async_copy(pt_hbm.at[:, pl.ds(u, TB)], pu_v.at[j], sem)
                pltpu.async_copy(qt_hbm.at[:, pl.ds(i, TB)], qi_v.at[j], sem)
            for j in range(SUB):
                pltpu.make_async_copy(
                    pt_hbm.at[:, pl.ds(0, TB)], pu_v.at[j], sem).wait()
                pltpu.make_async_copy(
                    qt_hbm.at[:, pl.ds(0, TB)], qi_v.at[j], sem).wait()

            keep = (lane >> 3) == half

            def kstep(k, a):
                kvec = jnp.full((L,), 0, jnp.int32) + k
                pv = plsc.load_gather(pu_v, [blk, kvec, ucol])
                qv = plsc.load_gather(qi_v, [blk, kvec, icol])
                return jnp.where(keep, a + pv * qv, a)

            acc = lax.fori_loop(0, K, kstep, acc)

        o_v[pl.ds(g * L, L)] = acc
        return carry

    lax.fori_loop(0, BPW // L, pair16, 0)

    pltpu.sync_copy(o_v, out_hbm.at[pl.ds(base, BPW)])


def _gmf(user_ids, item_ids, P, Q):
    mesh = plsc.VectorSubcoreMesh(
        core_axis_name="c", subcore_axis_name="s",
        num_cores=NC, num_subcores=NS)
    run = pl.kernel(
        _gmf_body,
        out_type=jax.ShapeDtypeStruct((BATCH,), jnp.float32),
        mesh=mesh,
        compiler_params=pltpu.CompilerParams(needs_layout_passes=False),
        scratch_types=[
            pltpu.VMEM((BPW + SUB,), jnp.int32),
            pltpu.VMEM((BPW + SUB,), jnp.int32),
            pltpu.VMEM((SUB, K, TB), jnp.float32),
            pltpu.VMEM((SUB, K, TB), jnp.float32),
            pltpu.VMEM((BPW,), jnp.float32),
            pltpu.SemaphoreType.DMA,
        ],
    )
    return run(user_ids, item_ids,
               jnp.swapaxes(P, 0, 1), jnp.swapaxes(Q, 0, 1))


def kernel(user_ids, item_ids, P, Q):
    out = _gmf(user_ids, item_ids, P, Q)
    return out.reshape(BATCH, 1)
